# baseline (speedup 1.0000x reference)
import functools

import jax
import jax.numpy as jnp
from jax import lax
from jax.experimental import pallas as pl
from jax.experimental.pallas import tpu as pltpu
from jax.experimental.pallas import tpu_sc as plsc

N = 10000
E = 320000
CH = 128
HALF = CH // 2

NC = 2
NS = 16
NW = NC * NS

LANES = 16
CHUNK = 128
EPT = 20480
STEPS = EPT // CHUNK
GROUP = 32
E_PAD = EPT * NS
NPAD = 10240
ROWS_PER_SUB = NPAD // NS


def _mm_body(x_ref, w_ref, b_ref, o_ref):
    acc = jnp.dot(x_ref[...], w_ref[...], preferred_element_type=jnp.float32)
    acc = jnp.maximum(acc + b_ref[...], 0.0)
    o_ref[0] = acc[:, :HALF]
    o_ref[1] = acc[:, HALF:]


def _tc_project(x, w, b):
    grid = (NPAD // 2048,)
    return pl.pallas_call(
        _mm_body,
        grid=grid,
        in_specs=[
            pl.BlockSpec((2048, CH), lambda i: (i, 0)),
            pl.BlockSpec((CH, CH), lambda i: (0, 0)),
            pl.BlockSpec((1, CH), lambda i: (0, 0)),
        ],
        out_specs=pl.BlockSpec((2, 2048, HALF), lambda i: (0, i, 0)),
        out_shape=jax.ShapeDtypeStruct((2, NPAD, HALF), jnp.float32),
    )(x, w, b)


def _spmm_body(src_hbm, dst_hbm, val_hbm, base_hbm, b0_hbm, b1_hbm,
               out_hbm,
               src_v, dst_v, val_v, rows0_v, rows1_v, rows2_v, rows3_v,
               base_sh, acc_sh,
               sem0, sem1, sem2, sem3, ssem0, ssem1, ssem2, ssem3):
    cid = lax.axis_index("c")
    sid = lax.axis_index("s")

    stripe = pl.ds(sid * ROWS_PER_SUB, ROWS_PER_SUB)
    pltpu.sync_copy(base_hbm.at[cid, stripe], base_sh.at[stripe])

    zeros16 = jnp.zeros((LANES,), jnp.float32)

    def _zrow(r, _):
        for j in range(HALF // LANES):
            rows0_v[r, pl.ds(j * LANES, LANES)] = zeros16
        return 0

    lax.fori_loop(0, CHUNK, _zrow, 0)
    for blk in range(ROWS_PER_SUB // CHUNK):
        pltpu.sync_copy(
            rows0_v, acc_sh.at[pl.ds(sid * ROWS_PER_SUB + blk * CHUNK, CHUNK)])
    plsc.subcore_barrier()

    def _scale(rows_ref, k):
        def _grp(g, _):
            vvec = val_v[k, pl.ds(g * LANES, LANES)]
            for l in range(LANES):
                bcast = vvec.at[jnp.full((LANES,), l, jnp.int32)].get(
                    mode="promise_in_bounds")
                e = g * LANES + l
                for j in range(HALF // LANES):
                    sl = pl.ds(j * LANES, LANES)
                    rows_ref[e, sl] = rows_ref[e, sl] * bcast
            return 0

        lax.fori_loop(0, CHUNK // LANES, _grp, 0)

    bufs = (rows0_v, rows1_v, rows2_v, rows3_v)
    gsems = (sem0, sem1, sem2, sem3)
    ssems = (ssem0, ssem1, ssem2, ssem3)

    def _gather(k, rows_ref, sem):
        return pltpu.async_copy(base_sh.at[dst_v.at[k]], rows_ref, sem)

    def _gather_hbm(k, rows_ref, sem):
        @pl.when(cid == 0)
        def _():
            pltpu.async_copy(b0_hbm.at[dst_v.at[k]], rows_ref, sem)

        @pl.when(cid == 1)
        def _():
            pltpu.async_copy(b1_hbm.at[dst_v.at[k]], rows_ref, sem)

    def _gwait(rows_ref, sem):
        pltpu.make_async_copy(base_sh.at[dst_v.at[0]], rows_ref, sem).wait()

    def _scatter(k, rows_ref, sem):
        return pltpu.async_copy(rows_ref, acc_sh.at[src_v.at[k]], sem,
                                add=True)

    def _swait(rows_ref, sem):
        pltpu.make_async_copy(rows_ref, acc_sh.at[src_v.at[0]], sem).wait()

    def _group(gi, _):
        goff = sid * STEPS + gi * GROUP
        pltpu.sync_copy(src_hbm.at[pl.ds(goff, GROUP)], src_v)
        pltpu.sync_copy(dst_hbm.at[pl.ds(goff, GROUP)], dst_v)
        pltpu.sync_copy(val_hbm.at[pl.ds(goff, GROUP)], val_v)
        _gather(0, rows0_v, sem0)
        _gather(1, rows1_v, sem1)

        def _quad(q, _):
            for j in range(4):
                k = 4 * q + j
                j2 = (j + 2) % 4
                _gwait(bufs[j], gsems[j])
                _scale(bufs[j], k)
                _scatter(k, bufs[j], ssems[j])

                @pl.when(jnp.logical_and(k + 2 < GROUP, k >= 2))
                def _():
                    _swait(bufs[j2], ssems[j2])

                @pl.when(k + 2 < GROUP)
                def _():
                    if j2 == 3:
                        _gather_hbm(k + 2, bufs[j2], gsems[j2])
                    else:
                        _gather(k + 2, bufs[j2], gsems[j2])

            return 0

        lax.fori_loop(0, GROUP // 4, _quad, 0)
        for j in range(4):
            _swait(bufs[j], ssems[j])
        return 0

    lax.fori_loop(0, STEPS // GROUP, _group, 0)
    plsc.subcore_barrier()

    pltpu.sync_copy(acc_sh.at[stripe], out_hbm.at[cid, stripe])


_sc_spmm = functools.partial(
    pl.kernel,
    out_type=jax.ShapeDtypeStruct((NC, NPAD, HALF), jnp.float32),
    mesh=plsc.VectorSubcoreMesh(core_axis_name="c", subcore_axis_name="s"),
    compiler_params=pltpu.CompilerParams(use_tc_tiling_on_sc=False),
    scratch_types=[
        pltpu.VMEM((GROUP, CHUNK), jnp.int32),
        pltpu.VMEM((GROUP, CHUNK), jnp.int32),
        pltpu.VMEM((GROUP, CHUNK), jnp.float32),
        pltpu.VMEM((CHUNK, HALF), jnp.float32),
        pltpu.VMEM((CHUNK, HALF), jnp.float32),
        pltpu.VMEM((CHUNK, HALF), jnp.float32),
        pltpu.VMEM((CHUNK, HALF), jnp.float32),
        pltpu.VMEM_SHARED((NPAD, HALF), jnp.float32),
        pltpu.VMEM_SHARED((NPAD, HALF), jnp.float32),
        pltpu.SemaphoreType.DMA,
        pltpu.SemaphoreType.DMA,
        pltpu.SemaphoreType.DMA,
        pltpu.SemaphoreType.DMA,
        pltpu.SemaphoreType.DMA,
        pltpu.SemaphoreType.DMA,
        pltpu.SemaphoreType.DMA,
        pltpu.SemaphoreType.DMA,
    ],
)(_spmm_body)


@jax.jit
def kernel(adj_index, adj_values, features, W, b):
    src = adj_index[0]
    dst = adj_index[1]
    pad = E_PAD - E
    src_p = jnp.concatenate([src, jnp.zeros((pad,), jnp.int32)]
                            ).reshape(NS * STEPS, CHUNK)
    dst_p = jnp.concatenate([dst, jnp.zeros((pad,), jnp.int32)]
                            ).reshape(NS * STEPS, CHUNK)
    val_p = jnp.concatenate([adj_values, jnp.zeros((pad,), jnp.float32)]
                            ).reshape(NS * STEPS, CHUNK)
    feat_p = jnp.pad(features, ((0, NPAD - N), (0, 0)))

    halves = _tc_project(feat_p, W, b)
    for _ in range(2):
        halves = _sc_spmm(src_p, dst_p, val_p, halves,
                          halves[0], halves[1])
    return jnp.concatenate([halves[0], halves[1]], axis=1)[:N]


# speedup vs baseline: 1.1174x; 1.1174x over previous
import functools

import jax
import jax.numpy as jnp
from jax import lax
from jax.experimental import pallas as pl
from jax.experimental.pallas import tpu as pltpu
from jax.experimental.pallas import tpu_sc as plsc

N = 10000
E = 320000
CH = 128
HALF = CH // 2

NC = 2
NS = 16
NW = NC * NS

LANES = 16
CHUNK = 128
EPT = 20480
STEPS = EPT // CHUNK
GROUP = 16
E_PAD = EPT * NS
NPAD = 10240
ROWS_PER_SUB = NPAD // NS


def _mm_body(x_ref, w_ref, b_ref, o_ref):
    acc = jnp.dot(x_ref[...], w_ref[...], preferred_element_type=jnp.float32)
    acc = jnp.maximum(acc + b_ref[...], 0.0)
    o_ref[0] = acc[:, :HALF]
    o_ref[1] = acc[:, HALF:]


def _tc_project(x, w, b):
    grid = (NPAD // 2048,)
    return pl.pallas_call(
        _mm_body,
        grid=grid,
        in_specs=[
            pl.BlockSpec((2048, CH), lambda i: (i, 0)),
            pl.BlockSpec((CH, CH), lambda i: (0, 0)),
            pl.BlockSpec((1, CH), lambda i: (0, 0)),
        ],
        out_specs=pl.BlockSpec((2, 2048, HALF), lambda i: (0, i, 0)),
        out_shape=jax.ShapeDtypeStruct((2, NPAD, HALF), jnp.float32),
    )(x, w, b)


def _spmm_body(src_hbm, dst_hbm, val_hbm, base_hbm, out_hbm,
               src_a, dst_a, val_a, src_b, dst_b, val_b,
               rows0_v, rows1_v, rows2_v, rows3_v,
               base_sh, acc_sh,
               sem0, sem1, sem2, sem3, ssem0, ssem1, ssem2, ssem3,
               isem_a, isem_b):
    cid = lax.axis_index("c")
    sid = lax.axis_index("s")

    stripe = pl.ds(sid * ROWS_PER_SUB, ROWS_PER_SUB)
    pltpu.sync_copy(base_hbm.at[cid, stripe], base_sh.at[stripe])

    zeros16 = jnp.zeros((LANES,), jnp.float32)

    def _zrow(r, _):
        for j in range(HALF // LANES):
            rows0_v[r, pl.ds(j * LANES, LANES)] = zeros16
        return 0

    lax.fori_loop(0, CHUNK, _zrow, 0)
    for blk in range(ROWS_PER_SUB // CHUNK):
        pltpu.sync_copy(
            rows0_v, acc_sh.at[pl.ds(sid * ROWS_PER_SUB + blk * CHUNK, CHUNK)])
    plsc.subcore_barrier()

    def _scale(val_v, rows_ref, k):
        def _grp(g, _):
            vvec = val_v[k, pl.ds(g * LANES, LANES)]
            for l in range(LANES):
                bcast = vvec.at[jnp.full((LANES,), l, jnp.int32)].get(
                    mode="promise_in_bounds")
                e = g * LANES + l
                for j in range(HALF // LANES):
                    sl = pl.ds(j * LANES, LANES)
                    rows_ref[e, sl] = rows_ref[e, sl] * bcast
            return 0

        lax.fori_loop(0, CHUNK // LANES, _grp, 0)

    bufs = (rows0_v, rows1_v, rows2_v, rows3_v)
    gsems = (sem0, sem1, sem2, sem3)
    ssems = (ssem0, ssem1, ssem2, ssem3)

    def _gather(dst_v, k, rows_ref, sem):
        return pltpu.async_copy(base_sh.at[dst_v.at[k]], rows_ref, sem)

    def _gwait(rows_ref, sem):
        pltpu.make_async_copy(base_sh.at[dst_a.at[0]], rows_ref, sem).wait()

    def _scatter(src_v, k, rows_ref, sem):
        return pltpu.async_copy(rows_ref, acc_sh.at[src_v.at[k]], sem,
                                add=True)

    def _swait(rows_ref, sem):
        pltpu.make_async_copy(rows_ref, acc_sh.at[src_a.at[0]], sem).wait()

    def _refill(goff, sv, dv, vv, isem):
        pltpu.async_copy(src_hbm.at[pl.ds(goff, GROUP)], sv, isem)
        pltpu.async_copy(dst_hbm.at[pl.ds(goff, GROUP)], dv, isem)
        pltpu.async_copy(val_hbm.at[pl.ds(goff, GROUP)], vv, isem)

    def _refill_wait(sv, dv, vv, isem):
        z = pl.ds(0, GROUP)
        pltpu.make_async_copy(src_hbm.at[z], sv, isem).wait()
        pltpu.make_async_copy(dst_hbm.at[z], dv, isem).wait()
        pltpu.make_async_copy(val_hbm.at[z], vv, isem).wait()

    def _run_group(gi, sv, dv, vv, isem, nsv, ndv, nvv, nisem):
        _refill_wait(sv, dv, vv, isem)

        @pl.when(gi + 1 < STEPS // GROUP)
        def _():
            _refill(sid * STEPS + (gi + 1) * GROUP, nsv, ndv, nvv, nisem)

        _gather(dv, 0, rows0_v, sem0)
        _gather(dv, 1, rows1_v, sem1)

        def _quad(q, _):
            for j in range(4):
                k = 4 * q + j
                j2 = (j + 2) % 4
                _gwait(bufs[j], gsems[j])
                _scale(vv, bufs[j], k)
                _scatter(sv, k, bufs[j], ssems[j])

                @pl.when(jnp.logical_and(k + 2 < GROUP, k >= 2))
                def _():
                    _swait(bufs[j2], ssems[j2])

                @pl.when(k + 2 < GROUP)
                def _():
                    _gather(dv, k + 2, bufs[j2], gsems[j2])

            return 0

        lax.fori_loop(0, GROUP // 4, _quad, 0)
        for j in range(4):
            _swait(bufs[j], ssems[j])

    _refill(sid * STEPS, src_a, dst_a, val_a, isem_a)

    def _gpair(h, _):
        _run_group(2 * h, src_a, dst_a, val_a, isem_a,
                   src_b, dst_b, val_b, isem_b)
        _run_group(2 * h + 1, src_b, dst_b, val_b, isem_b,
                   src_a, dst_a, val_a, isem_a)
        return 0

    lax.fori_loop(0, STEPS // GROUP // 2, _gpair, 0)
    plsc.subcore_barrier()

    pltpu.sync_copy(acc_sh.at[stripe], out_hbm.at[cid, stripe])


_sc_spmm = functools.partial(
    pl.kernel,
    out_type=jax.ShapeDtypeStruct((NC, NPAD, HALF), jnp.float32),
    mesh=plsc.VectorSubcoreMesh(core_axis_name="c", subcore_axis_name="s"),
    compiler_params=pltpu.CompilerParams(use_tc_tiling_on_sc=False),
    scratch_types=[
        pltpu.VMEM((GROUP, CHUNK), jnp.int32),
        pltpu.VMEM((GROUP, CHUNK), jnp.int32),
        pltpu.VMEM((GROUP, CHUNK), jnp.float32),
        pltpu.VMEM((GROUP, CHUNK), jnp.int32),
        pltpu.VMEM((GROUP, CHUNK), jnp.int32),
        pltpu.VMEM((GROUP, CHUNK), jnp.float32),
        pltpu.VMEM((CHUNK, HALF), jnp.float32),
        pltpu.VMEM((CHUNK, HALF), jnp.float32),
        pltpu.VMEM((CHUNK, HALF), jnp.float32),
        pltpu.VMEM((CHUNK, HALF), jnp.float32),
        pltpu.VMEM_SHARED((NPAD, HALF), jnp.float32),
        pltpu.VMEM_SHARED((NPAD, HALF), jnp.float32),
        pltpu.SemaphoreType.DMA,
        pltpu.SemaphoreType.DMA,
        pltpu.SemaphoreType.DMA,
        pltpu.SemaphoreType.DMA,
        pltpu.SemaphoreType.DMA,
        pltpu.SemaphoreType.DMA,
        pltpu.SemaphoreType.DMA,
        pltpu.SemaphoreType.DMA,
        pltpu.SemaphoreType.DMA,
        pltpu.SemaphoreType.DMA,
    ],
)(_spmm_body)


@jax.jit
def kernel(adj_index, adj_values, features, W, b):
    src = adj_index[0]
    dst = adj_index[1]
    pad = E_PAD - E
    src_p = jnp.concatenate([src, jnp.zeros((pad,), jnp.int32)]
                            ).reshape(NS * STEPS, CHUNK)
    dst_p = jnp.concatenate([dst, jnp.zeros((pad,), jnp.int32)]
                            ).reshape(NS * STEPS, CHUNK)
    val_p = jnp.concatenate([adj_values, jnp.zeros((pad,), jnp.float32)]
                            ).reshape(NS * STEPS, CHUNK)
    feat_p = jnp.pad(features, ((0, NPAD - N), (0, 0)))

    halves = _tc_project(feat_p, W, b)
    for _ in range(2):
        halves = _sc_spmm(src_p, dst_p, val_p, halves)
    return jnp.concatenate([halves[0], halves[1]], axis=1)[:N]


# speedup vs baseline: 1.1316x; 1.0128x over previous
import functools

import jax
import jax.numpy as jnp
from jax import lax
from jax.experimental import pallas as pl
from jax.experimental.pallas import tpu as pltpu
from jax.experimental.pallas import tpu_sc as plsc

N = 10000
E = 320000
CH = 128
HALF = CH // 2

NC = 2
NS = 16
NW = NC * NS

LANES = 16
CHUNK = 128
EPT = 20480
STEPS = EPT // CHUNK
GROUP = 16
E_PAD = EPT * NS
NPAD = 10240
ROWS_PER_SUB = NPAD // NS


def _mm_body(x_ref, w_ref, b_ref, o_ref):
    acc = jnp.dot(x_ref[...], w_ref[...], preferred_element_type=jnp.float32)
    acc = jnp.maximum(acc + b_ref[...], 0.0)
    o_ref[0] = acc[:, :HALF]
    o_ref[1] = acc[:, HALF:]


def _tc_project(x, w, b):
    grid = (NPAD // 2048,)
    return pl.pallas_call(
        _mm_body,
        grid=grid,
        in_specs=[
            pl.BlockSpec((2048, CH), lambda i: (i, 0)),
            pl.BlockSpec((CH, CH), lambda i: (0, 0)),
            pl.BlockSpec((1, CH), lambda i: (0, 0)),
        ],
        out_specs=pl.BlockSpec((2, 2048, HALF), lambda i: (0, i, 0)),
        out_shape=jax.ShapeDtypeStruct((2, NPAD, HALF), jnp.float32),
    )(x, w, b)


def _spmm_body(src_hbm, dst_hbm, val_hbm, base_hbm, out_hbm,
               src_a, dst_a, val_a, src_b, dst_b, val_b,
               rows0_v, rows1_v, rows2_v, rows3_v,
               base_sh, acc_sh,
               sem0, sem1, sem2, sem3, ssem0, ssem1, ssem2, ssem3,
               isem_a, isem_b):
    cid = lax.axis_index("c")
    sid = lax.axis_index("s")

    stripe = pl.ds(sid * ROWS_PER_SUB, ROWS_PER_SUB)
    pltpu.sync_copy(base_hbm.at[cid, stripe], base_sh.at[stripe])

    zeros16 = jnp.zeros((LANES,), jnp.float32)

    def _zero_stripe(tgt_sh):
        def _zrow(r, _):
            for j in range(HALF // LANES):
                rows0_v[r, pl.ds(j * LANES, LANES)] = zeros16
            return 0

        lax.fori_loop(0, CHUNK, _zrow, 0)
        for blk in range(ROWS_PER_SUB // CHUNK):
            pltpu.sync_copy(
                rows0_v,
                tgt_sh.at[pl.ds(sid * ROWS_PER_SUB + blk * CHUNK, CHUNK)])

    _zero_stripe(acc_sh)
    plsc.subcore_barrier()

    def _scale(val_v, rows_ref, k):
        def _grp(g, _):
            vvec = val_v[k, pl.ds(g * LANES, LANES)]
            for l in range(LANES):
                bcast = vvec.at[jnp.full((LANES,), l, jnp.int32)].get(
                    mode="promise_in_bounds")
                e = g * LANES + l
                for j in range(HALF // LANES):
                    sl = pl.ds(j * LANES, LANES)
                    rows_ref[e, sl] = rows_ref[e, sl] * bcast
            return 0

        lax.fori_loop(0, CHUNK // LANES, _grp, 0)

    bufs = (rows0_v, rows1_v, rows2_v, rows3_v)
    gsems = (sem0, sem1, sem2, sem3)
    ssems = (ssem0, ssem1, ssem2, ssem3)

    def _gather(gref, dst_v, k, rows_ref, sem):
        return pltpu.async_copy(gref.at[dst_v.at[k]], rows_ref, sem)

    def _gwait(rows_ref, sem):
        pltpu.make_async_copy(base_sh.at[dst_a.at[0]], rows_ref, sem).wait()

    def _scatter(aref, src_v, k, rows_ref, sem):
        return pltpu.async_copy(rows_ref, aref.at[src_v.at[k]], sem,
                                add=True)

    def _swait(rows_ref, sem):
        pltpu.make_async_copy(rows_ref, acc_sh.at[src_a.at[0]], sem).wait()

    def _refill(goff, sv, dv, vv, isem):
        pltpu.async_copy(src_hbm.at[pl.ds(goff, GROUP)], sv, isem)
        pltpu.async_copy(dst_hbm.at[pl.ds(goff, GROUP)], dv, isem)
        pltpu.async_copy(val_hbm.at[pl.ds(goff, GROUP)], vv, isem)

    def _refill_wait(sv, dv, vv, isem):
        z = pl.ds(0, GROUP)
        pltpu.make_async_copy(src_hbm.at[z], sv, isem).wait()
        pltpu.make_async_copy(dst_hbm.at[z], dv, isem).wait()
        pltpu.make_async_copy(val_hbm.at[z], vv, isem).wait()

    def _run_group(gref, aref, gi, sv, dv, vv, isem, nsv, ndv, nvv, nisem):
        _refill_wait(sv, dv, vv, isem)

        @pl.when(gi + 1 < STEPS // GROUP)
        def _():
            _refill(sid * STEPS + (gi + 1) * GROUP, nsv, ndv, nvv, nisem)

        _gather(gref, dv, 0, rows0_v, sem0)
        _gather(gref, dv, 1, rows1_v, sem1)

        def _quad(q, _):
            for j in range(4):
                k = 4 * q + j
                j2 = (j + 2) % 4
                _gwait(bufs[j], gsems[j])
                _scale(vv, bufs[j], k)
                _scatter(aref, sv, k, bufs[j], ssems[j])

                @pl.when(jnp.logical_and(k + 2 < GROUP, k >= 2))
                def _():
                    _swait(bufs[j2], ssems[j2])

                @pl.when(k + 2 < GROUP)
                def _():
                    _gather(gref, dv, k + 2, bufs[j2], gsems[j2])

            return 0

        lax.fori_loop(0, GROUP // 4, _quad, 0)
        for j in range(4):
            _swait(bufs[j], ssems[j])

    def _round(gref, aref):
        _refill(sid * STEPS, src_a, dst_a, val_a, isem_a)

        def _gpair(h, _):
            _run_group(gref, aref, 2 * h, src_a, dst_a, val_a, isem_a,
                       src_b, dst_b, val_b, isem_b)
            _run_group(gref, aref, 2 * h + 1, src_b, dst_b, val_b, isem_b,
                       src_a, dst_a, val_a, isem_a)
            return 0

        lax.fori_loop(0, STEPS // GROUP // 2, _gpair, 0)
        plsc.subcore_barrier()

    _round(base_sh, acc_sh)
    _zero_stripe(base_sh)
    plsc.subcore_barrier()
    _round(acc_sh, base_sh)

    pltpu.sync_copy(base_sh.at[stripe], out_hbm.at[cid, stripe])


_sc_spmm = functools.partial(
    pl.kernel,
    out_type=jax.ShapeDtypeStruct((NC, NPAD, HALF), jnp.float32),
    mesh=plsc.VectorSubcoreMesh(core_axis_name="c", subcore_axis_name="s"),
    compiler_params=pltpu.CompilerParams(use_tc_tiling_on_sc=False),
    scratch_types=[
        pltpu.VMEM((GROUP, CHUNK), jnp.int32),
        pltpu.VMEM((GROUP, CHUNK), jnp.int32),
        pltpu.VMEM((GROUP, CHUNK), jnp.float32),
        pltpu.VMEM((GROUP, CHUNK), jnp.int32),
        pltpu.VMEM((GROUP, CHUNK), jnp.int32),
        pltpu.VMEM((GROUP, CHUNK), jnp.float32),
        pltpu.VMEM((CHUNK, HALF), jnp.float32),
        pltpu.VMEM((CHUNK, HALF), jnp.float32),
        pltpu.VMEM((CHUNK, HALF), jnp.float32),
        pltpu.VMEM((CHUNK, HALF), jnp.float32),
        pltpu.VMEM_SHARED((NPAD, HALF), jnp.float32),
        pltpu.VMEM_SHARED((NPAD, HALF), jnp.float32),
        pltpu.SemaphoreType.DMA,
        pltpu.SemaphoreType.DMA,
        pltpu.SemaphoreType.DMA,
        pltpu.SemaphoreType.DMA,
        pltpu.SemaphoreType.DMA,
        pltpu.SemaphoreType.DMA,
        pltpu.SemaphoreType.DMA,
        pltpu.SemaphoreType.DMA,
        pltpu.SemaphoreType.DMA,
        pltpu.SemaphoreType.DMA,
    ],
)(_spmm_body)


@jax.jit
def kernel(adj_index, adj_values, features, W, b):
    src = adj_index[0]
    dst = adj_index[1]
    pad = E_PAD - E
    src_p = jnp.concatenate([src, jnp.zeros((pad,), jnp.int32)]
                            ).reshape(NS * STEPS, CHUNK)
    dst_p = jnp.concatenate([dst, jnp.zeros((pad,), jnp.int32)]
                            ).reshape(NS * STEPS, CHUNK)
    val_p = jnp.concatenate([adj_values, jnp.zeros((pad,), jnp.float32)]
                            ).reshape(NS * STEPS, CHUNK)
    feat_p = jnp.pad(features, ((0, NPAD - N), (0, 0)))

    halves = _tc_project(feat_p, W, b)
    halves = _sc_spmm(src_p, dst_p, val_p, halves)
    return jnp.concatenate([halves[0], halves[1]], axis=1)[:N]
